# Initial kernel scaffold; baseline (speedup 1.0000x reference)
import jax
import jax.numpy as jnp
from jax.experimental import pallas as pl


def kernel(dense, sparse, sparse_table, dense_W, dense_b, bias):
    raise NotImplementedError("write your pallas kernel here")


# speedup vs baseline: 1.2197x; 1.2197x over previous
import functools

import jax
import jax.numpy as jnp
from jax import lax
from jax.experimental import pallas as pl
from jax.experimental.pallas import tpu as pltpu
from jax.experimental.pallas import tpu_sc as plsc

B = 16384
F = 26
ND = 13
NC = 2
NS = 16
L = 16
NW = NC * NS
BPW = B // NW
CH = 128
NCHUNK = (BPW * F) // CH
KFIRE = 8

_mesh = plsc.VectorSubcoreMesh(core_axis_name="c", subcore_axis_name="s")


@functools.partial(
    pl.kernel,
    out_type=jax.ShapeDtypeStruct((B,), jnp.float32),
    mesh=_mesh,
    scratch_types=[
        pltpu.VMEM((NCHUNK, CH), jnp.int32),
        pltpu.VMEM((NCHUNK, CH), jnp.float32),
        pltpu.VMEM((ND, BPW), jnp.float32),
        pltpu.VMEM((ND + 2, L), jnp.float32),
        pltpu.VMEM((BPW,), jnp.float32),
        pltpu.SemaphoreType.DMA,
    ],
)
def _lr_kernel(table_hbm, idx_hbm, dense_hbm, w_hbm, out_hbm,
               idx_v, vals_v, dense_v, w_v, acc_v, sem):
    w = lax.axis_index("s") * NC + lax.axis_index("c")

    pltpu.sync_copy(idx_hbm.at[w], idx_v)
    pltpu.sync_copy(dense_hbm.at[w], dense_v)
    pltpu.sync_copy(w_hbm, w_v)

    def fire_group(g, carry):
        cps = [
            pltpu.async_copy(
                table_hbm.at[idx_v.at[g * KFIRE + t]],
                vals_v.at[g * KFIRE + t],
                sem,
            )
            for t in range(KFIRE)
        ]
        for cp in cps:
            cp.wait()
        return carry

    lax.fori_loop(0, NCHUNK // KFIRE, fire_group, 0)

    bvec = w_v[ND] + w_v[ND + 1]

    def row_block(i, carry):
        def f_add(f, a):
            off = f * BPW + i * L
            return a + vals_v[off // CH, pl.ds(off % CH, L)]

        a = lax.fori_loop(0, F, f_add, bvec)

        def d_add(d, a):
            return a + dense_v[d, pl.ds(i * L, L)] * w_v[d]

        a = lax.fori_loop(0, ND, d_add, a)
        acc_v[pl.ds(i * L, L)] = a
        return carry

    lax.fori_loop(0, BPW // L, row_block, 0)

    pltpu.sync_copy(acc_v, out_hbm.at[pl.ds(w * BPW, BPW)])


def kernel(dense, sparse, sparse_table, dense_W, dense_b, bias):
    table_flat = sparse_table.reshape(-1)
    idx = (
        sparse.astype(jnp.int32)
        .reshape(NW, BPW, F)
        .transpose(0, 2, 1)
        .reshape(NW, NCHUNK, CH)
    )
    dense_prep = dense.reshape(NW, BPW, ND).transpose(0, 2, 1)
    w_prep = jnp.concatenate(
        [
            jnp.broadcast_to(dense_W.reshape(ND, 1), (ND, L)),
            jnp.broadcast_to(dense_b.reshape(1, 1), (1, L)),
            jnp.broadcast_to(bias.reshape(1, 1), (1, L)),
        ],
        axis=0,
    )
    return _lr_kernel(table_flat, idx, dense_prep, w_prep)


# speedup vs baseline: 1.3718x; 1.1246x over previous
import functools

import jax
import jax.numpy as jnp
from jax import lax
from jax.experimental import pallas as pl
from jax.experimental.pallas import tpu as pltpu
from jax.experimental.pallas import tpu_sc as plsc

B = 16384
F = 26
ND = 13
NC = 2
NS = 16
L = 16
NW = NC * NS
BPW = B // NW
CH = 128
NCHUNK = (BPW * F) // CH
KFIRE = 8

_mesh = plsc.VectorSubcoreMesh(core_axis_name="c", subcore_axis_name="s")


@functools.partial(
    pl.kernel,
    out_type=jax.ShapeDtypeStruct((B,), jnp.float32),
    mesh=_mesh,
    scratch_types=[
        pltpu.VMEM((NCHUNK, CH), jnp.int32),
        pltpu.VMEM((NCHUNK, CH), jnp.float32),
        pltpu.VMEM((ND, BPW), jnp.float32),
        pltpu.VMEM((ND + 2, L), jnp.float32),
        pltpu.VMEM((BPW,), jnp.float32),
        pltpu.SemaphoreType.DMA,
    ],
)
def _lr_kernel(table_hbm, idx_hbm, dense_hbm, w_hbm, out_hbm,
               idx_v, vals_v, dense_v, w_v, acc_v, sem):
    w = lax.axis_index("s") * NC + lax.axis_index("c")

    pltpu.sync_copy(idx_hbm.at[w], idx_v)
    pltpu.sync_copy(dense_hbm.at[w], dense_v)
    pltpu.sync_copy(w_hbm, w_v)

    def fire_group(g, carry):
        for t in range(KFIRE):
            pltpu.async_copy(
                table_hbm.at[idx_v.at[g * KFIRE + t]],
                vals_v.at[g * KFIRE + t],
                sem,
            )
        return carry

    lax.fori_loop(0, NCHUNK // KFIRE, fire_group, 0)

    def drain(j, carry):
        pltpu.make_async_copy(
            table_hbm.at[idx_v.at[j]], vals_v.at[j], sem
        ).wait()
        return carry

    lax.fori_loop(0, NCHUNK, drain, 0)

    bvec = w_v[ND] + w_v[ND + 1]
    rpc = CH // L

    def row_block(i, carry):
        ro = i // rpc
        c = (i % rpc) * L
        a = bvec
        for f in range(F):
            a = a + vals_v[f * (BPW // CH) + ro, pl.ds(c, L)]
        dsl = pl.ds(i * L, L)
        for d in range(ND):
            a = a + dense_v[d, dsl] * w_v[d]
        acc_v[dsl] = a
        return carry

    lax.fori_loop(0, BPW // L, row_block, 0)

    pltpu.sync_copy(acc_v, out_hbm.at[pl.ds(w * BPW, BPW)])


def kernel(dense, sparse, sparse_table, dense_W, dense_b, bias):
    table_flat = sparse_table.reshape(-1)
    idx = (
        sparse.astype(jnp.int32)
        .reshape(NW, BPW, F)
        .transpose(0, 2, 1)
        .reshape(NW, NCHUNK, CH)
    )
    dense_prep = dense.reshape(NW, BPW, ND).transpose(0, 2, 1)
    w_prep = jnp.concatenate(
        [
            jnp.broadcast_to(dense_W.reshape(ND, 1), (ND, L)),
            jnp.broadcast_to(dense_b.reshape(1, 1), (1, L)),
            jnp.broadcast_to(bias.reshape(1, 1), (1, L)),
        ],
        axis=0,
    )
    return _lr_kernel(table_flat, idx, dense_prep, w_prep)
